# Initial kernel scaffold; baseline (speedup 1.0000x reference)
import jax
import jax.numpy as jnp
from jax.experimental import pallas as pl


def kernel(features, labels, prototypes):
    raise NotImplementedError("write your pallas kernel here")


# speedup vs baseline: 1.5058x; 1.5058x over previous
import jax
import jax.numpy as jnp
from jax.experimental import pallas as pl
from jax.experimental.pallas import tpu as pltpu

BATCH = 16384
FEAT = 128
NCLS = 1000
P1_BLK = 1024
P1_STEPS = BATCH // P1_BLK
P2_BLK = 256
P2_STEPS = BATCH // P2_BLK
MOM = 0.99


def _norm_rows(x):
    s2 = jnp.sum(x * x, axis=1, keepdims=True)
    return x * jax.lax.rsqrt(jnp.maximum(s2, 1e-24))


def _body(f1_ref, lab_ref, protos_ref, f2_ref, out_ref,
          sums_ref, counts_ref, pbf_ref):
    i = pl.program_id(0)

    @pl.when(i == 0)
    def _init():
        sums_ref[...] = jnp.zeros_like(sums_ref)
        counts_ref[...] = jnp.zeros_like(counts_ref)

    @pl.when(i < P1_STEPS)
    def _phase1():
        x = f1_ref[...]
        fn = _norm_rows(x).astype(jnp.bfloat16)
        lr = lab_ref[0]
        cls = jax.lax.broadcasted_iota(jnp.int32, (NCLS, P1_BLK), 0)
        oh = (cls == lr).astype(jnp.bfloat16)
        sums_ref[...] += jax.lax.dot_general(
            oh, fn, (((1,), (0,)), ((), ())),
            preferred_element_type=jnp.float32)
        counts_ref[...] += jnp.sum(
            oh.astype(jnp.float32), axis=1, keepdims=True)

    @pl.when(i == P1_STEPS)
    def _make_protos():
        p0 = _norm_rows(protos_ref[...])
        sn = _norm_rows(sums_ref[...])
        bl = _norm_rows(MOM * p0 + (1.0 - MOM) * sn)
        p = jnp.where(counts_ref[...] > 0.0, bl, p0)
        pbf_ref[...] = p.astype(jnp.bfloat16)

    @pl.when(i >= P1_STEPS)
    def _phase2():
        x = f2_ref[...]
        fn = _norm_rows(x).astype(jnp.bfloat16)
        s = jax.lax.dot_general(
            fn, pbf_ref[...], (((1,), (1,)), ((), ())),
            preferred_element_type=jnp.float32)
        out_ref[...] = -jnp.sqrt(jnp.maximum(2.0 - 2.0 * s, 0.0))


def kernel(features, labels, prototypes):
    labels3 = labels.reshape(P1_STEPS, 1, P1_BLK)
    grid = (P1_STEPS + P2_STEPS,)
    out = pl.pallas_call(
        _body,
        grid=grid,
        in_specs=[
            pl.BlockSpec((P1_BLK, FEAT), lambda i: (jnp.minimum(i, P1_STEPS - 1), 0)),
            pl.BlockSpec((1, 1, P1_BLK), lambda i: (jnp.minimum(i, P1_STEPS - 1), 0, 0)),
            pl.BlockSpec((NCLS, FEAT), lambda i: (0, 0)),
            pl.BlockSpec((P2_BLK, FEAT), lambda i: (jnp.maximum(i - P1_STEPS, 0), 0)),
        ],
        out_specs=pl.BlockSpec((P2_BLK, NCLS), lambda i: (jnp.maximum(i - P1_STEPS, 0), 0)),
        out_shape=jax.ShapeDtypeStruct((BATCH, NCLS), jnp.float32),
        scratch_shapes=[
            pltpu.VMEM((NCLS, FEAT), jnp.float32),
            pltpu.VMEM((NCLS, 1), jnp.float32),
            pltpu.VMEM((NCLS, FEAT), jnp.bfloat16),
        ],
    )(features, labels3, prototypes, features)
    return out


# speedup vs baseline: 1.6293x; 1.0820x over previous
import jax
import jax.numpy as jnp
from jax.experimental import pallas as pl
from jax.experimental.pallas import tpu as pltpu

BATCH = 16384
FEAT = 128
NCLS = 1000
P1_BLK = 1024
P1_STEPS = BATCH // P1_BLK
P2_BLK = 256
P2_STEPS = BATCH // P2_BLK
MOM = 0.99


def _norm_rows(x):
    s2 = jnp.sum(x * x, axis=1, keepdims=True)
    return x * jax.lax.rsqrt(jnp.maximum(s2, 1e-24))


def _body(f1_ref, lab_ref, protos_ref, out_ref,
          sums_ref, counts_ref, pbf_ref, fn_ref):
    i = pl.program_id(0)

    @pl.when(i == 0)
    def _init():
        sums_ref[...] = jnp.zeros_like(sums_ref)
        counts_ref[...] = jnp.zeros_like(counts_ref)

    @pl.when(i < P1_STEPS)
    def _phase1():
        x = f1_ref[...]
        fn = _norm_rows(x).astype(jnp.bfloat16)
        fn_ref[pl.ds(i * P1_BLK, P1_BLK), :] = fn
        lr = lab_ref[0]
        cls = jax.lax.broadcasted_iota(jnp.int32, (NCLS, P1_BLK), 0)
        oh = (cls == lr).astype(jnp.bfloat16)
        sums_ref[...] += jax.lax.dot_general(
            oh, fn, (((1,), (0,)), ((), ())),
            preferred_element_type=jnp.float32)
        counts_ref[...] += jnp.sum(
            oh.astype(jnp.float32), axis=1, keepdims=True)

    @pl.when(i == P1_STEPS)
    def _make_protos():
        p0 = _norm_rows(protos_ref[...])
        sn = _norm_rows(sums_ref[...])
        bl = _norm_rows(MOM * p0 + (1.0 - MOM) * sn)
        p = jnp.where(counts_ref[...] > 0.0, bl, p0)
        pbf_ref[...] = p.astype(jnp.bfloat16)

    @pl.when(i >= P1_STEPS)
    def _phase2():
        j = i - P1_STEPS
        fn = fn_ref[pl.ds(j * P2_BLK, P2_BLK), :]
        s = jax.lax.dot_general(
            fn, pbf_ref[...], (((1,), (1,)), ((), ())),
            preferred_element_type=jnp.float32)
        out_ref[...] = -jnp.sqrt(jnp.maximum(2.0 - 2.0 * s, 0.0))


def kernel(features, labels, prototypes):
    labels3 = labels.reshape(P1_STEPS, 1, P1_BLK)
    grid = (P1_STEPS + P2_STEPS,)
    out = pl.pallas_call(
        _body,
        grid=grid,
        in_specs=[
            pl.BlockSpec((P1_BLK, FEAT), lambda i: (jnp.minimum(i, P1_STEPS - 1), 0)),
            pl.BlockSpec((1, 1, P1_BLK), lambda i: (jnp.minimum(i, P1_STEPS - 1), 0, 0)),
            pl.BlockSpec((NCLS, FEAT), lambda i: (0, 0)),
        ],
        out_specs=pl.BlockSpec((P2_BLK, NCLS), lambda i: (jnp.maximum(i - P1_STEPS, 0), 0)),
        out_shape=jax.ShapeDtypeStruct((BATCH, NCLS), jnp.float32),
        scratch_shapes=[
            pltpu.VMEM((NCLS, FEAT), jnp.float32),
            pltpu.VMEM((NCLS, 1), jnp.float32),
            pltpu.VMEM((NCLS, FEAT), jnp.bfloat16),
            pltpu.VMEM((BATCH, FEAT), jnp.bfloat16),
        ],
    )(features, labels3, prototypes)
    return out


# speedup vs baseline: 1.9567x; 1.2009x over previous
import jax
import jax.numpy as jnp
from jax.experimental import pallas as pl
from jax.experimental.pallas import tpu as pltpu

BATCH = 16384
FEAT = 128
NCLS = 1000
P1_BLK = 1024
P1_STEPS = BATCH // P1_BLK
P2_BLK = 2048
P2_STEPS = BATCH // P2_BLK
MOM = 0.99


def _norm_rows(x):
    s2 = jnp.sum(x * x, axis=1, keepdims=True)
    return x * jax.lax.rsqrt(jnp.maximum(s2, 1e-24))


def _body(f1_ref, lab_ref, protos_ref, out_ref,
          sums_ref, counts_ref, pbf_ref, fn_ref):
    i = pl.program_id(0)

    @pl.when(i == 0)
    def _init():
        sums_ref[...] = jnp.zeros_like(sums_ref)
        counts_ref[...] = jnp.zeros_like(counts_ref)

    @pl.when(i < P1_STEPS)
    def _phase1():
        x = f1_ref[...]
        fn = _norm_rows(x).astype(jnp.bfloat16)
        fn_ref[pl.ds(i * P1_BLK, P1_BLK), :] = fn
        lr = lab_ref[0]
        cls = jax.lax.broadcasted_iota(jnp.int32, (NCLS, P1_BLK), 0)
        oh = (cls == lr).astype(jnp.bfloat16)
        sums_ref[...] += jax.lax.dot_general(
            oh, fn, (((1,), (0,)), ((), ())),
            preferred_element_type=jnp.float32)
        counts_ref[...] += jnp.sum(
            oh.astype(jnp.float32), axis=1, keepdims=True)

    @pl.when(i == P1_STEPS)
    def _make_protos():
        p0 = _norm_rows(protos_ref[...])
        sn = _norm_rows(sums_ref[...])
        bl = _norm_rows(MOM * p0 + (1.0 - MOM) * sn)
        p = jnp.where(counts_ref[...] > 0.0, bl, p0)
        pbf_ref[...] = p.astype(jnp.bfloat16)

    @pl.when(i >= P1_STEPS)
    def _phase2():
        j = i - P1_STEPS
        fn = fn_ref[pl.ds(j * P2_BLK, P2_BLK), :]
        s = jax.lax.dot_general(
            fn, pbf_ref[...], (((1,), (1,)), ((), ())),
            preferred_element_type=jnp.float32)
        out_ref[...] = -jnp.sqrt(jnp.maximum(2.0 - 2.0 * s, 0.0))


def kernel(features, labels, prototypes):
    labels3 = labels.reshape(P1_STEPS, 1, P1_BLK)
    grid = (P1_STEPS + P2_STEPS,)
    out = pl.pallas_call(
        _body,
        grid=grid,
        in_specs=[
            pl.BlockSpec((P1_BLK, FEAT), lambda i: (jnp.minimum(i, P1_STEPS - 1), 0)),
            pl.BlockSpec((1, 1, P1_BLK), lambda i: (jnp.minimum(i, P1_STEPS - 1), 0, 0)),
            pl.BlockSpec((NCLS, FEAT), lambda i: (0, 0)),
        ],
        out_specs=pl.BlockSpec((P2_BLK, NCLS), lambda i: (jnp.maximum(i - P1_STEPS, 0), 0)),
        out_shape=jax.ShapeDtypeStruct((BATCH, NCLS), jnp.float32),
        scratch_shapes=[
            pltpu.VMEM((NCLS, FEAT), jnp.float32),
            pltpu.VMEM((NCLS, 1), jnp.float32),
            pltpu.VMEM((NCLS, FEAT), jnp.bfloat16),
            pltpu.VMEM((BATCH, FEAT), jnp.bfloat16),
        ],
    )(features, labels3, prototypes)
    return out
